# baseline (speedup 1.0000x reference)
import functools

import jax
import jax.numpy as jnp
import numpy as np
from jax import lax
from jax.experimental import pallas as pl
from jax.experimental.pallas import tpu as pltpu
from jax.experimental.pallas import tpu_sc as plsc

B = 2
S = 4096
HIDDEN = 2048
H = 32
D = 64
NH = 2
NB = 128
CL = 64
NC = (NH * S) // CL
N = NH * S
NKEY = NH * NB
DP = 2 * D


def _proj_body(a_ref, wqk_ref, wv_ref, out_ref):
    a = a_ref[...]
    qk = jnp.dot(a, wqk_ref[...], preferred_element_type=jnp.float32,
                 precision=jax.lax.Precision.HIGHEST)
    v = jnp.dot(a, wv_ref[...], preferred_element_type=jnp.float32,
                precision=jax.lax.Precision.HIGHEST)
    bm, bn = qk.shape
    out_ref[:, :, 0, :] = qk.reshape(bm, bn // D, D)
    out_ref[:, :, 1, :] = v.reshape(bm, bn // D, D)


def _projections(hs, Wqk, Wv):
    a = hs.reshape(B * S, HIDDEN)
    bm, bn = 512, 512
    grid = (HIDDEN // bn, B * S // bm)
    qkv = pl.pallas_call(
        _proj_body,
        grid=grid,
        in_specs=[
            pl.BlockSpec((bm, HIDDEN), lambda n, m: (m, 0)),
            pl.BlockSpec((HIDDEN, bn), lambda n, m: (0, n)),
            pl.BlockSpec((HIDDEN, bn), lambda n, m: (0, n)),
        ],
        out_specs=pl.BlockSpec((bm, bn // D, NH, D), lambda n, m: (m, n, 0, 0)),
        out_shape=jax.ShapeDtypeStruct((B * S, H, 2, D), jnp.float32),
        compiler_params=pltpu.CompilerParams(
            dimension_semantics=("parallel", "parallel"),
        ),
    )(a, Wqk, Wv)
    return qkv


def _hash_body(qkv_ref, rot_ref, out_ref):
    rot = rot_ref[...]
    for h in range(H):
        q = qkv_ref[:, h, 0, :]
        r = jnp.dot(q, rot, preferred_element_type=jnp.float32,
                    precision=jax.lax.Precision.HIGHEST)
        out_ref[0, h, 0, :] = jnp.argmax(r[:, :NB], axis=-1).astype(jnp.int32)
        out_ref[0, h, 1, :] = jnp.argmax(r[:, NB:], axis=-1).astype(jnp.int32) + NB


def _hash_buckets(qkv):
    TB = 256
    TPB = S // TB
    grid = (B * TPB,)
    buckets = pl.pallas_call(
        _hash_body,
        grid=grid,
        in_specs=[
            pl.BlockSpec((TB, H, NH, D), lambda m: (m, 0, 0, 0)),
            pl.BlockSpec((D, NKEY), lambda m: (0, 0)),
        ],
        out_specs=pl.BlockSpec((1, H, NH, TB), lambda m: (m // TPB, 0, 0, m % TPB)),
        out_shape=jax.ShapeDtypeStruct((B, H, NH, S), jnp.int32),
        compiler_params=pltpu.CompilerParams(
            dimension_semantics=("parallel",),
        ),
    )(qkv, jnp.asarray(_hash_rotations()))
    return buckets


def _hash_rotations():
    rng = np.random.RandomState(0)
    rot = np.asarray(rng.normal(size=(D, NH, NB // 2)), dtype=np.float32)
    return np.concatenate(
        [rot[:, 0], -rot[:, 0], rot[:, 1], -rot[:, 1]], axis=1)


_NLANE = 16
_SEG = N // _NLANE


def _sc_sort_body(bk_hbm, qkv_hbm, qkvs_hbm, ts_hbm, invp_hbm,
                  bk_v, hist_v, start_v, tot_v, gst_v, dest_v, invp_v, ts_v,
                  gidx_v, rows_v, rows2_v, sem, sem2):
    wid = lax.axis_index("s") * 2 + lax.axis_index("c")
    lane = lax.iota(jnp.int32, _NLANE)

    def do_pair(pp, _c):
        p = wid * 2 + pp
        pltpu.sync_copy(bk_hbm.at[p], bk_v)

        def zero(i, _):
            hist_v[pl.ds(i * 16, 16)] = jnp.zeros((16,), jnp.int32)
            return 0
        lax.fori_loop(0, NKEY * _NLANE // 16, zero, 0)

        def pa(s_, _):
            idx = lane * _SEG + s_
            vb = plsc.load_gather(bk_v, [idx])
            ha = lane * NKEY + vb
            h = plsc.load_gather(hist_v, [ha])
            plsc.store_scatter(hist_v, [ha], h + 1)
            return 0
        lax.fori_loop(0, _SEG, pa, 0)

        def pb(j, _):
            acc = jnp.zeros((16,), jnp.int32)
            for l in range(_NLANE):
                off = l * NKEY + j * 16
                row = hist_v[pl.ds(off, 16)]
                start_v[pl.ds(off, 16)] = acc
                acc = acc + row
            tot_v[pl.ds(j * 16, 16)] = acc
            return 0
        lax.fori_loop(0, NKEY // 16, pb, 0)

        def pc(j, carry):
            vv = tot_v[pl.ds(j * 16, 16)]
            inc = plsc.cumsum(vv)
            gst_v[pl.ds(j * 16, 16)] = inc - vv + carry
            return carry + jnp.sum(vv)
        lax.fori_loop(0, NKEY // 16, pc, jnp.int32(0))

        def pd(j, _):
            g = gst_v[pl.ds(j * 16, 16)]
            for l in range(_NLANE):
                off = l * NKEY + j * 16
                start_v[pl.ds(off, 16)] = start_v[pl.ds(off, 16)] + g
            return 0
        lax.fori_loop(0, NKEY // 16, pd, 0)

        def pe(s_, _):
            idx = lane * _SEG + s_
            vb = plsc.load_gather(bk_v, [idx])
            ha = lane * NKEY + vb
            ptr = plsc.load_gather(start_v, [ha])
            plsc.store_scatter(start_v, [ha], ptr + 1)
            plsc.store_scatter(dest_v, [idx], ptr)
            return 0
        lax.fori_loop(0, _SEG, pe, 0)

        def pf(s_, _):
            d = dest_v[pl.ds(s_ * 16, 16)]
            plsc.store_scatter(invp_v, [d], s_ * 16 + lane)
            return 0
        lax.fori_loop(0, N // 16, pf, 0)

        base = (p // H) * (S * H) + (p % H)
        def pg(s_, _):
            iv = invp_v[pl.ds(s_ * 16, 16)]
            st = jnp.bitwise_and(iv, S - 1)
            ts_v[pl.ds(s_ * 16, 16)] = st
            gidx_v[pl.ds(s_ * 16, 16)] = st * H + base
            return 0
        lax.fori_loop(0, N // 16, pg, 0)
        pltpu.sync_copy(ts_v, ts_hbm.at[p])
        pltpu.sync_copy(invp_v, invp_hbm.at[p])

        def ph(j, _):
            i1 = gidx_v.at[pl.ds(j * 256, 128)]
            i2 = gidx_v.at[pl.ds(j * 256 + 128, 128)]
            cp1 = pltpu.async_copy(qkv_hbm.at[i1], rows_v, sem)
            cp2 = pltpu.async_copy(qkv_hbm.at[i2], rows2_v, sem2)
            cp1.wait()
            cp2.wait()
            pltpu.sync_copy(rows_v, qkvs_hbm.at[pl.ds(p * N + j * 256, 128)])
            pltpu.sync_copy(rows2_v, qkvs_hbm.at[pl.ds(p * N + j * 256 + 128, 128)])
            return 0
        lax.fori_loop(0, N // 256, ph, 0)
        return 0

    lax.fori_loop(0, 2, do_pair, 0)


def _sc_sort_gather(bk2, qkv_t):
    f = pl.kernel(
        _sc_sort_body,
        out_type=[
            jax.ShapeDtypeStruct((B * H * N, DP), jnp.float32),
            jax.ShapeDtypeStruct((B * H, N), jnp.int32),
            jax.ShapeDtypeStruct((B * H, N), jnp.int32),
        ],
        mesh=plsc.VectorSubcoreMesh(core_axis_name="c", subcore_axis_name="s"),
        compiler_params=pltpu.CompilerParams(needs_layout_passes=False),
        scratch_types=[
            pltpu.VMEM((N,), jnp.int32),
            pltpu.VMEM((NKEY * _NLANE,), jnp.int32),
            pltpu.VMEM((NKEY * _NLANE,), jnp.int32),
            pltpu.VMEM((NKEY,), jnp.int32),
            pltpu.VMEM((NKEY,), jnp.int32),
            pltpu.VMEM((N,), jnp.int32),
            pltpu.VMEM((N,), jnp.int32),
            pltpu.VMEM((N,), jnp.int32),
            pltpu.VMEM((N,), jnp.int32),
            pltpu.VMEM((128, DP), jnp.float32),
            pltpu.VMEM((128, DP), jnp.float32),
            pltpu.SemaphoreType.DMA,
            pltpu.SemaphoreType.DMA,
        ],
    )
    return f(bk2, qkv_t)


NG = 32
GROUPS = NC // NG


def _attn_body(qkv_ref, qkvp_ref, t_ref, tp_ref, out_ref):
    cur = qkv_ref[0][:, :, 0:D]
    vcur = qkv_ref[0][:, :, D:DP]
    prevg = qkvp_ref[0][:, :, 0:D]
    vprevg = qkvp_ref[0][:, :, D:DP]
    tcur = t_ref[0]
    tprevg = tp_ref[0][NG - 1:NG]

    def norm(x):
        var = jnp.mean(x * x, axis=-1, keepdims=True)
        return x * jax.lax.rsqrt(var + 1e-6) * np.float32(1.0 / np.sqrt(D))

    kcur = norm(cur)
    kprev = jnp.concatenate([norm(prevg), kcur[: NG - 1]], axis=0)
    vprev = jnp.concatenate([vprevg, vcur[: NG - 1]], axis=0)
    tprev = jnp.concatenate([tprevg, tcur[: NG - 1]], axis=0)

    k2 = jnp.concatenate([kprev, kcur], axis=1)
    v2 = jnp.concatenate([vprev, vcur], axis=1)
    t2 = jnp.concatenate([tprev, tcur], axis=1)

    dots = jax.lax.dot_general(cur, k2, (((2,), (2,)), ((0,), (0,))),
                               preferred_element_type=jnp.float32)
    tq = tcur[:, :, None]
    tk = t2[:, None, :]
    dots = dots - (tq < tk).astype(jnp.float32) * 1e9 \
                - (tq == tk).astype(jnp.float32) * 1e5
    mx = jnp.max(dots, axis=-1)
    p = jnp.exp(dots - mx[:, :, None])
    ssum = jnp.sum(p, axis=-1)
    lg = mx + jnp.log(ssum)
    o = jax.lax.dot_general(p, v2, (((2,), (1,)), ((0,), (0,))),
                            preferred_element_type=jnp.float32)
    out_ref[0, :, :, 0:D] = o / ssum[:, :, None]
    out_ref[0, :, :, D:DP] = jnp.broadcast_to(lg[:, :, None], (NG, CL, D))


def _attention(qkvs, t_s):
    grid = (B * H, GROUPS)
    blk = lambda p, g: (p, g, 0, 0)
    blkp = lambda p, g: (p, (g * NG - 1) % NC, 0, 0)
    blk3 = lambda p, g: (p, g, 0)
    blk3p = lambda p, g: (p, (g + GROUPS - 1) % GROUPS, 0)
    out = pl.pallas_call(
        _attn_body,
        grid=grid,
        in_specs=[
            pl.BlockSpec((1, NG, CL, DP), blk),
            pl.BlockSpec((1, 1, CL, DP), blkp),
            pl.BlockSpec((1, NG, CL), blk3),
            pl.BlockSpec((1, NG, CL), blk3p),
        ],
        out_specs=pl.BlockSpec((1, NG, CL, DP), blk),
        out_shape=jax.ShapeDtypeStruct((B * H, NC, CL, DP), jnp.float32),
        compiler_params=pltpu.CompilerParams(
            dimension_semantics=("parallel", "arbitrary"),
        ),
    )(qkvs, qkvs, t_s, t_s)
    return out


def _sc_unsort_body(outs_hbm, invp_hbm, outu_hbm,
                    invp_v, sidx_v, rows_v, sem):
    wid = lax.axis_index("s") * 2 + lax.axis_index("c")
    lane = lax.iota(jnp.int32, _NLANE)

    def do_pair(pp, _c):
        p = wid * 2 + pp
        pltpu.sync_copy(invp_hbm.at[p], invp_v)

        def pa(s_, _):
            iv = invp_v[pl.ds(s_ * 16, 16)]
            plsc.store_scatter(
                sidx_v,
                [jnp.full((16,), s_ // 8, jnp.int32), (s_ % 8) * 16 + lane],
                iv + p * N)
            return 0
        lax.fori_loop(0, N // 16, pa, 0)

        def pb(j, _):
            pltpu.sync_copy(outs_hbm.at[pl.ds(p * N + j * 128, 128)], rows_v)
            pltpu.async_copy(rows_v, outu_hbm.at[sidx_v.at[j]], sem).wait()
            return 0
        lax.fori_loop(0, N // 128, pb, 0)
        return 0

    lax.fori_loop(0, 2, do_pair, 0)


def _sc_unsort(outs, invp):
    f = pl.kernel(
        _sc_unsort_body,
        out_type=jax.ShapeDtypeStruct((B * H * N, DP), jnp.float32),
        mesh=plsc.VectorSubcoreMesh(core_axis_name="c", subcore_axis_name="s"),
        compiler_params=pltpu.CompilerParams(needs_layout_passes=False),
        scratch_types=[
            pltpu.VMEM((N,), jnp.int32),
            pltpu.VMEM((N // 128, 128), jnp.int32),
            pltpu.VMEM((128, DP), jnp.float32),
            pltpu.SemaphoreType.DMA,
        ],
    )
    return f(outs, invp)


_TBC = 128


def _comb_body(ou_ref, out_ref):
    ou = ou_ref[0]
    o0 = ou[:, 0, :, 0:D]
    o1 = ou[:, 1, :, 0:D]
    lg0 = ou[:, 0, :, D]
    lg1 = ou[:, 1, :, D]
    m = jnp.maximum(lg0, lg1)
    e0 = jnp.exp(lg0 - m)
    e1 = jnp.exp(lg1 - m)
    w0 = (e0 / (e0 + e1))[:, :, None]
    w1 = (e1 / (e0 + e1))[:, :, None]
    o = o0 * w0 + o1 * w1
    out_ref[0] = o.transpose(1, 0, 2).reshape(_TBC, H * D)


def _combine(outu):
    grid = (B, S // _TBC)
    return pl.pallas_call(
        _comb_body,
        grid=grid,
        in_specs=[
            pl.BlockSpec((1, H, NH, _TBC, DP), lambda b, t: (b, 0, 0, t, 0)),
        ],
        out_specs=pl.BlockSpec((1, _TBC, H * D), lambda b, t: (b, t, 0)),
        out_shape=jax.ShapeDtypeStruct((B, S, H * D), jnp.float32),
        compiler_params=pltpu.CompilerParams(
            dimension_semantics=("parallel", "parallel"),
        ),
    )(outu)


def kernel(hidden_states, Wqk, Wv):
    qkv = _projections(hidden_states, Wqk, Wv)
    buckets = _hash_buckets(qkv)

    bk2 = buckets.reshape(B * H, N)
    qkv_t = qkv.reshape(B * S * H, DP)
    qkvs, ts, invp = _sc_sort_gather(bk2, qkv_t)

    outs = _attention(
        qkvs.reshape(B * H, NC, CL, DP),
        ts.reshape(B * H, NC, CL),
    )

    outu = _sc_unsort(outs.reshape(B * H * N, DP), invp)
    return _combine(outu.reshape(B, H, NH, S, DP))


# speedup vs baseline: 1.2812x; 1.2812x over previous
import functools

import jax
import jax.numpy as jnp
import numpy as np
from jax import lax
from jax.experimental import pallas as pl
from jax.experimental.pallas import tpu as pltpu
from jax.experimental.pallas import tpu_sc as plsc

B = 2
S = 4096
HIDDEN = 2048
H = 32
D = 64
NH = 2
NB = 128
CL = 64
NC = (NH * S) // CL
N = NH * S
NKEY = NH * NB
DP = 2 * D


def _proj_body(a_ref, wqk_ref, wv_ref, out_ref):
    a = a_ref[...]
    qk = jnp.dot(a, wqk_ref[...], preferred_element_type=jnp.float32)
    v = jnp.dot(a, wv_ref[...], preferred_element_type=jnp.float32)
    bm, bn = qk.shape
    out_ref[:, :, 0, :] = qk.reshape(bm, bn // D, D)
    out_ref[:, :, 1, :] = v.reshape(bm, bn // D, D)


def _projections(hs, Wqk, Wv):
    a = hs.reshape(B * S, HIDDEN)
    bm, bn = 512, 512
    grid = (HIDDEN // bn, B * S // bm)
    qkv = pl.pallas_call(
        _proj_body,
        grid=grid,
        in_specs=[
            pl.BlockSpec((bm, HIDDEN), lambda n, m: (m, 0)),
            pl.BlockSpec((HIDDEN, bn), lambda n, m: (0, n)),
            pl.BlockSpec((HIDDEN, bn), lambda n, m: (0, n)),
        ],
        out_specs=pl.BlockSpec((bm, bn // D, NH, D), lambda n, m: (m, n, 0, 0)),
        out_shape=jax.ShapeDtypeStruct((B * S, H, 2, D), jnp.float32),
        compiler_params=pltpu.CompilerParams(
            dimension_semantics=("parallel", "parallel"),
        ),
    )(a, Wqk, Wv)
    return qkv


def _hash_body(qkv_ref, rot_ref, out_ref):
    rot = rot_ref[...]
    for h in range(H):
        q = qkv_ref[:, h, 0, :]
        r = jnp.dot(q, rot, preferred_element_type=jnp.float32)
        out_ref[0, h, 0, :] = jnp.argmax(r[:, :NB], axis=-1).astype(jnp.int32)
        out_ref[0, h, 1, :] = jnp.argmax(r[:, NB:], axis=-1).astype(jnp.int32) + NB


def _hash_buckets(qkv):
    TB = 256
    TPB = S // TB
    grid = (B * TPB,)
    buckets = pl.pallas_call(
        _hash_body,
        grid=grid,
        in_specs=[
            pl.BlockSpec((TB, H, NH, D), lambda m: (m, 0, 0, 0)),
            pl.BlockSpec((D, NKEY), lambda m: (0, 0)),
        ],
        out_specs=pl.BlockSpec((1, H, NH, TB), lambda m: (m // TPB, 0, 0, m % TPB)),
        out_shape=jax.ShapeDtypeStruct((B, H, NH, S), jnp.int32),
        compiler_params=pltpu.CompilerParams(
            dimension_semantics=("parallel",),
        ),
    )(qkv, jnp.asarray(_hash_rotations()))
    return buckets


def _hash_rotations():
    rng = np.random.RandomState(0)
    rot = np.asarray(rng.normal(size=(D, NH, NB // 2)), dtype=np.float32)
    return np.concatenate(
        [rot[:, 0], -rot[:, 0], rot[:, 1], -rot[:, 1]], axis=1)


_NLANE = 16
_SEG = N // _NLANE


def _sc_sort_body(bk_hbm, qkv_hbm, qkvs_hbm, ts_hbm, invp_hbm,
                  bk_v, hist_v, start_v, tot_v, gst_v, dest_v, invp_v, ts_v,
                  gidx_v, rows_v, rows2_v, sem, sem2):
    wid = lax.axis_index("s") * 2 + lax.axis_index("c")
    lane = lax.iota(jnp.int32, _NLANE)

    def do_pair(pp, _c):
        p = wid * 2 + pp
        pltpu.sync_copy(bk_hbm.at[p], bk_v)

        def zero(i, _):
            hist_v[pl.ds(i * 16, 16)] = jnp.zeros((16,), jnp.int32)
            return 0
        lax.fori_loop(0, NKEY * _NLANE // 16, zero, 0)

        def pa(s_, _):
            idx = lane * _SEG + s_
            vb = plsc.load_gather(bk_v, [idx])
            ha = lane * NKEY + vb
            h = plsc.load_gather(hist_v, [ha])
            plsc.store_scatter(hist_v, [ha], h + 1)
            return 0
        lax.fori_loop(0, _SEG, pa, 0)

        def pb(j, _):
            acc = jnp.zeros((16,), jnp.int32)
            for l in range(_NLANE):
                off = l * NKEY + j * 16
                row = hist_v[pl.ds(off, 16)]
                start_v[pl.ds(off, 16)] = acc
                acc = acc + row
            tot_v[pl.ds(j * 16, 16)] = acc
            return 0
        lax.fori_loop(0, NKEY // 16, pb, 0)

        def pc(j, carry):
            vv = tot_v[pl.ds(j * 16, 16)]
            inc = plsc.cumsum(vv)
            gst_v[pl.ds(j * 16, 16)] = inc - vv + carry
            return carry + jnp.sum(vv)
        lax.fori_loop(0, NKEY // 16, pc, jnp.int32(0))

        def pd(j, _):
            g = gst_v[pl.ds(j * 16, 16)]
            for l in range(_NLANE):
                off = l * NKEY + j * 16
                start_v[pl.ds(off, 16)] = start_v[pl.ds(off, 16)] + g
            return 0
        lax.fori_loop(0, NKEY // 16, pd, 0)

        def pe(s_, _):
            idx = lane * _SEG + s_
            vb = plsc.load_gather(bk_v, [idx])
            ha = lane * NKEY + vb
            ptr = plsc.load_gather(start_v, [ha])
            plsc.store_scatter(start_v, [ha], ptr + 1)
            plsc.store_scatter(dest_v, [idx], ptr)
            return 0
        lax.fori_loop(0, _SEG, pe, 0)

        def pf(s_, _):
            d = dest_v[pl.ds(s_ * 16, 16)]
            plsc.store_scatter(invp_v, [d], s_ * 16 + lane)
            return 0
        lax.fori_loop(0, N // 16, pf, 0)

        base = (p // H) * (S * H) + (p % H)
        def pg(s_, _):
            iv = invp_v[pl.ds(s_ * 16, 16)]
            st = jnp.bitwise_and(iv, S - 1)
            ts_v[pl.ds(s_ * 16, 16)] = st
            gidx_v[pl.ds(s_ * 16, 16)] = st * H + base
            return 0
        lax.fori_loop(0, N // 16, pg, 0)
        pltpu.sync_copy(ts_v, ts_hbm.at[p])
        pltpu.sync_copy(invp_v, invp_hbm.at[p])

        def ph(j, _):
            i1 = gidx_v.at[pl.ds(j * 256, 128)]
            i2 = gidx_v.at[pl.ds(j * 256 + 128, 128)]
            cp1 = pltpu.async_copy(qkv_hbm.at[i1], rows_v, sem)
            cp2 = pltpu.async_copy(qkv_hbm.at[i2], rows2_v, sem2)
            cp1.wait()
            cp2.wait()
            pltpu.sync_copy(rows_v, qkvs_hbm.at[pl.ds(p * N + j * 256, 128)])
            pltpu.sync_copy(rows2_v, qkvs_hbm.at[pl.ds(p * N + j * 256 + 128, 128)])
            return 0
        lax.fori_loop(0, N // 256, ph, 0)
        return 0

    lax.fori_loop(0, 2, do_pair, 0)


def _sc_sort_gather(bk2, qkv_t):
    f = pl.kernel(
        _sc_sort_body,
        out_type=[
            jax.ShapeDtypeStruct((B * H * N, DP), jnp.float32),
            jax.ShapeDtypeStruct((B * H, N), jnp.int32),
            jax.ShapeDtypeStruct((B * H, N), jnp.int32),
        ],
        mesh=plsc.VectorSubcoreMesh(core_axis_name="c", subcore_axis_name="s"),
        compiler_params=pltpu.CompilerParams(needs_layout_passes=False),
        scratch_types=[
            pltpu.VMEM((N,), jnp.int32),
            pltpu.VMEM((NKEY * _NLANE,), jnp.int32),
            pltpu.VMEM((NKEY * _NLANE,), jnp.int32),
            pltpu.VMEM((NKEY,), jnp.int32),
            pltpu.VMEM((NKEY,), jnp.int32),
            pltpu.VMEM((N,), jnp.int32),
            pltpu.VMEM((N,), jnp.int32),
            pltpu.VMEM((N,), jnp.int32),
            pltpu.VMEM((N,), jnp.int32),
            pltpu.VMEM((128, DP), jnp.float32),
            pltpu.VMEM((128, DP), jnp.float32),
            pltpu.SemaphoreType.DMA,
            pltpu.SemaphoreType.DMA,
        ],
    )
    return f(bk2, qkv_t)


NG = 32
GROUPS = NC // NG


def _attn_body(qkv_ref, qkvp_ref, t_ref, tp_ref, out_ref):
    cur = qkv_ref[0][:, :, 0:D]
    vcur = qkv_ref[0][:, :, D:DP]
    prevg = qkvp_ref[0][:, :, 0:D]
    vprevg = qkvp_ref[0][:, :, D:DP]
    tcur = t_ref[0]
    tprevg = tp_ref[0][NG - 1:NG]

    def norm(x):
        var = jnp.mean(x * x, axis=-1, keepdims=True)
        return x * jax.lax.rsqrt(var + 1e-6) * np.float32(1.0 / np.sqrt(D))

    kcur = norm(cur)
    kprev = jnp.concatenate([norm(prevg), kcur[: NG - 1]], axis=0)
    vprev = jnp.concatenate([vprevg, vcur[: NG - 1]], axis=0)
    tprev = jnp.concatenate([tprevg, tcur[: NG - 1]], axis=0)

    k2 = jnp.concatenate([kprev, kcur], axis=1)
    v2 = jnp.concatenate([vprev, vcur], axis=1)
    t2 = jnp.concatenate([tprev, tcur], axis=1)

    dots = jax.lax.dot_general(cur, k2, (((2,), (2,)), ((0,), (0,))),
                               preferred_element_type=jnp.float32)
    tq = tcur[:, :, None]
    tk = t2[:, None, :]
    dots = dots - (tq < tk).astype(jnp.float32) * 1e9 \
                - (tq == tk).astype(jnp.float32) * 1e5
    mx = jnp.max(dots, axis=-1)
    p = jnp.exp(dots - mx[:, :, None])
    ssum = jnp.sum(p, axis=-1)
    lg = mx + jnp.log(ssum)
    o = jax.lax.dot_general(p, v2, (((2,), (1,)), ((0,), (0,))),
                            preferred_element_type=jnp.float32)
    out_ref[0, :, :, 0:D] = o / ssum[:, :, None]
    out_ref[0, :, :, D:DP] = jnp.broadcast_to(lg[:, :, None], (NG, CL, D))


def _attention(qkvs, t_s):
    grid = (B * H, GROUPS)
    blk = lambda p, g: (p, g, 0, 0)
    blkp = lambda p, g: (p, (g * NG - 1) % NC, 0, 0)
    blk3 = lambda p, g: (p, g, 0)
    blk3p = lambda p, g: (p, (g + GROUPS - 1) % GROUPS, 0)
    out = pl.pallas_call(
        _attn_body,
        grid=grid,
        in_specs=[
            pl.BlockSpec((1, NG, CL, DP), blk),
            pl.BlockSpec((1, 1, CL, DP), blkp),
            pl.BlockSpec((1, NG, CL), blk3),
            pl.BlockSpec((1, NG, CL), blk3p),
        ],
        out_specs=pl.BlockSpec((1, NG, CL, DP), blk),
        out_shape=jax.ShapeDtypeStruct((B * H, NC, CL, DP), jnp.float32),
        compiler_params=pltpu.CompilerParams(
            dimension_semantics=("parallel", "arbitrary"),
        ),
    )(qkvs, qkvs, t_s, t_s)
    return out


def _sc_unsort_body(outs_hbm, invp_hbm, outu_hbm,
                    invp_v, sidx_v, rows_v, sem):
    wid = lax.axis_index("s") * 2 + lax.axis_index("c")
    lane = lax.iota(jnp.int32, _NLANE)

    def do_pair(pp, _c):
        p = wid * 2 + pp
        pltpu.sync_copy(invp_hbm.at[p], invp_v)

        def pa(s_, _):
            iv = invp_v[pl.ds(s_ * 16, 16)]
            plsc.store_scatter(
                sidx_v,
                [jnp.full((16,), s_ // 8, jnp.int32), (s_ % 8) * 16 + lane],
                iv + p * N)
            return 0
        lax.fori_loop(0, N // 16, pa, 0)

        def pb(j, _):
            pltpu.sync_copy(outs_hbm.at[pl.ds(p * N + j * 128, 128)], rows_v)
            pltpu.async_copy(rows_v, outu_hbm.at[sidx_v.at[j]], sem).wait()
            return 0
        lax.fori_loop(0, N // 128, pb, 0)
        return 0

    lax.fori_loop(0, 2, do_pair, 0)


def _sc_unsort(outs, invp):
    f = pl.kernel(
        _sc_unsort_body,
        out_type=jax.ShapeDtypeStruct((B * H * N, DP), jnp.float32),
        mesh=plsc.VectorSubcoreMesh(core_axis_name="c", subcore_axis_name="s"),
        compiler_params=pltpu.CompilerParams(needs_layout_passes=False),
        scratch_types=[
            pltpu.VMEM((N,), jnp.int32),
            pltpu.VMEM((N // 128, 128), jnp.int32),
            pltpu.VMEM((128, DP), jnp.float32),
            pltpu.SemaphoreType.DMA,
        ],
    )
    return f(outs, invp)


_TBC = 128


def _comb_body(ou_ref, out_ref):
    ou = ou_ref[0]
    o0 = ou[:, 0, :, 0:D]
    o1 = ou[:, 1, :, 0:D]
    lg0 = ou[:, 0, :, D]
    lg1 = ou[:, 1, :, D]
    m = jnp.maximum(lg0, lg1)
    e0 = jnp.exp(lg0 - m)
    e1 = jnp.exp(lg1 - m)
    w0 = (e0 / (e0 + e1))[:, :, None]
    w1 = (e1 / (e0 + e1))[:, :, None]
    o = o0 * w0 + o1 * w1
    out_ref[0] = o.transpose(1, 0, 2).reshape(_TBC, H * D)


def _combine(outu):
    grid = (B, S // _TBC)
    return pl.pallas_call(
        _comb_body,
        grid=grid,
        in_specs=[
            pl.BlockSpec((1, H, NH, _TBC, DP), lambda b, t: (b, 0, 0, t, 0)),
        ],
        out_specs=pl.BlockSpec((1, _TBC, H * D), lambda b, t: (b, t, 0)),
        out_shape=jax.ShapeDtypeStruct((B, S, H * D), jnp.float32),
        compiler_params=pltpu.CompilerParams(
            dimension_semantics=("parallel", "parallel"),
        ),
    )(outu)


def kernel(hidden_states, Wqk, Wv):
    qkv = _projections(hidden_states, Wqk, Wv)
    buckets = _hash_buckets(qkv)

    bk2 = buckets.reshape(B * H, N)
    qkv_t = qkv.reshape(B * S * H, DP)
    qkvs, ts, invp = _sc_sort_gather(bk2, qkv_t)

    outs = _attention(
        qkvs.reshape(B * H, NC, CL, DP),
        ts.reshape(B * H, NC, CL),
    )

    outu = _sc_unsort(outs.reshape(B * H * N, DP), invp)
    return _combine(outu.reshape(B, H, NH, S, DP))


# speedup vs baseline: 1.2816x; 1.0003x over previous
import jax
import jax.numpy as jnp
import numpy as np
from jax import lax
from jax.experimental import pallas as pl
from jax.experimental.pallas import tpu as pltpu
from jax.experimental.pallas import tpu_sc as plsc

B = 2
S = 4096
HIDDEN = 2048
H = 32
D = 64
NH = 2
NB = 128
CL = 64
NC = (NH * S) // CL
N = NH * S
NKEY = NH * NB
DP = 2 * D


def _proj_body(a_ref, wqk_ref, wv_ref, out_ref):
    a = a_ref[...]
    qk = jnp.dot(a, wqk_ref[...], preferred_element_type=jnp.float32)
    v = jnp.dot(a, wv_ref[...], preferred_element_type=jnp.float32)
    bm, bn = qk.shape
    out_ref[:, :, 0, :] = qk.reshape(bm, bn // D, D)
    out_ref[:, :, 1, :] = v.reshape(bm, bn // D, D)


def _projections(hs, Wqk, Wv):
    a = hs.reshape(B * S, HIDDEN)
    bm, bn = 512, 512
    grid = (HIDDEN // bn, B * S // bm)
    qkv = pl.pallas_call(
        _proj_body,
        grid=grid,
        in_specs=[
            pl.BlockSpec((bm, HIDDEN), lambda n, m: (m, 0)),
            pl.BlockSpec((HIDDEN, bn), lambda n, m: (0, n)),
            pl.BlockSpec((HIDDEN, bn), lambda n, m: (0, n)),
        ],
        out_specs=pl.BlockSpec((bm, bn // D, NH, D), lambda n, m: (m, n, 0, 0)),
        out_shape=jax.ShapeDtypeStruct((B * S, H, 2, D), jnp.float32),
        compiler_params=pltpu.CompilerParams(
            dimension_semantics=("parallel", "parallel"),
        ),
    )(a, Wqk, Wv)
    return qkv


def _hash_body(qkv_ref, rot_ref, out_ref):
    rot = rot_ref[...]
    for h in range(H):
        q = qkv_ref[:, h, 0, :]
        r = jnp.dot(q, rot, preferred_element_type=jnp.float32)
        out_ref[0, h, 0, :] = jnp.argmax(r[:, :NB], axis=-1).astype(jnp.int32)
        out_ref[0, h, 1, :] = jnp.argmax(r[:, NB:], axis=-1).astype(jnp.int32) + NB


def _hash_buckets(qkv):
    TB = 256
    TPB = S // TB
    grid = (B * TPB,)
    buckets = pl.pallas_call(
        _hash_body,
        grid=grid,
        in_specs=[
            pl.BlockSpec((TB, H, NH, D), lambda m: (m, 0, 0, 0)),
            pl.BlockSpec((D, NKEY), lambda m: (0, 0)),
        ],
        out_specs=pl.BlockSpec((1, H, NH, TB), lambda m: (m // TPB, 0, 0, m % TPB)),
        out_shape=jax.ShapeDtypeStruct((B, H, NH, S), jnp.int32),
        compiler_params=pltpu.CompilerParams(
            dimension_semantics=("parallel",),
        ),
    )(qkv, jnp.asarray(_hash_rotations()))
    return buckets


def _hash_rotations():
    rng = np.random.RandomState(0)
    rot = np.asarray(rng.normal(size=(D, NH, NB // 2)), dtype=np.float32)
    return np.concatenate(
        [rot[:, 0], -rot[:, 0], rot[:, 1], -rot[:, 1]], axis=1)


_NLANE = 16
_SEG = N // _NLANE


def _sc_sort_body(bk_hbm, qkv_hbm, qkvs_hbm, ts_hbm, invp_hbm,
                  bk_v, hist_v, start_v, tot_v, gst_v, dest_v, invp_v, ts_v,
                  gidx_v, rows_v, rows2_v, sem, sem2):
    wid = lax.axis_index("s") * 2 + lax.axis_index("c")
    lane = lax.iota(jnp.int32, _NLANE)

    def do_pair(pp, _c):
        p = wid * 2 + pp
        pltpu.sync_copy(bk_hbm.at[p], bk_v)

        def zero(i, _):
            hist_v[pl.ds(i * 16, 16)] = jnp.zeros((16,), jnp.int32)
            return 0
        lax.fori_loop(0, NKEY * _NLANE // 16, zero, 0)

        def pa(s_, _):
            idx = lane * _SEG + s_
            vb = plsc.load_gather(bk_v, [idx])
            ha = lane * NKEY + vb
            h = plsc.load_gather(hist_v, [ha])
            plsc.store_scatter(hist_v, [ha], h + 1)
            return 0
        lax.fori_loop(0, _SEG, pa, 0)

        def pb(j, _):
            acc = jnp.zeros((16,), jnp.int32)
            for l in range(_NLANE):
                off = l * NKEY + j * 16
                row = hist_v[pl.ds(off, 16)]
                start_v[pl.ds(off, 16)] = acc
                acc = acc + row
            tot_v[pl.ds(j * 16, 16)] = acc
            return 0
        lax.fori_loop(0, NKEY // 16, pb, 0)

        def pc(j, carry):
            vv = tot_v[pl.ds(j * 16, 16)]
            inc = plsc.cumsum(vv)
            gst_v[pl.ds(j * 16, 16)] = inc - vv + carry
            return carry + jnp.sum(vv)
        lax.fori_loop(0, NKEY // 16, pc, jnp.int32(0))

        def pd(j, _):
            g = gst_v[pl.ds(j * 16, 16)]
            for l in range(_NLANE):
                off = l * NKEY + j * 16
                start_v[pl.ds(off, 16)] = start_v[pl.ds(off, 16)] + g
            return 0
        lax.fori_loop(0, NKEY // 16, pd, 0)

        def pe(s_, _):
            idx = lane * _SEG + s_
            vb = plsc.load_gather(bk_v, [idx])
            ha = lane * NKEY + vb
            ptr = plsc.load_gather(start_v, [ha])
            plsc.store_scatter(start_v, [ha], ptr + 1)
            plsc.store_scatter(dest_v, [idx], ptr)
            return 0
        lax.fori_loop(0, _SEG, pe, 0)

        def pf(s_, _):
            d = dest_v[pl.ds(s_ * 16, 16)]
            plsc.store_scatter(invp_v, [d], s_ * 16 + lane)
            return 0
        lax.fori_loop(0, N // 16, pf, 0)

        base = (p // H) * (S * H) + (p % H)
        def pg(s_, _):
            iv = invp_v[pl.ds(s_ * 16, 16)]
            st = jnp.bitwise_and(iv, S - 1)
            ts_v[pl.ds(s_ * 16, 16)] = st
            gidx_v[pl.ds(s_ * 16, 16)] = st * H + base
            return 0
        lax.fori_loop(0, N // 16, pg, 0)
        pltpu.sync_copy(ts_v, ts_hbm.at[p])
        pltpu.sync_copy(invp_v, invp_hbm.at[p])

        def ph(j, _):
            i1 = gidx_v.at[pl.ds(j * 256, 128)]
            i2 = gidx_v.at[pl.ds(j * 256 + 128, 128)]
            cp1 = pltpu.async_copy(qkv_hbm.at[i1], rows_v, sem)
            cp2 = pltpu.async_copy(qkv_hbm.at[i2], rows2_v, sem2)
            cp1.wait()
            cp2.wait()
            pltpu.sync_copy(rows_v, qkvs_hbm.at[pl.ds(p * N + j * 256, 128)])
            pltpu.sync_copy(rows2_v, qkvs_hbm.at[pl.ds(p * N + j * 256 + 128, 128)])
            return 0
        lax.fori_loop(0, N // 256, ph, 0)
        return 0

    lax.fori_loop(0, 2, do_pair, 0)


def _sc_sort_gather(bk2, qkv_t):
    f = pl.kernel(
        _sc_sort_body,
        out_type=[
            jax.ShapeDtypeStruct((B * H * N, DP), jnp.float32),
            jax.ShapeDtypeStruct((B * H, N), jnp.int32),
            jax.ShapeDtypeStruct((B * H, N), jnp.int32),
        ],
        mesh=plsc.VectorSubcoreMesh(core_axis_name="c", subcore_axis_name="s"),
        compiler_params=pltpu.CompilerParams(needs_layout_passes=False),
        scratch_types=[
            pltpu.VMEM((N,), jnp.int32),
            pltpu.VMEM((NKEY * _NLANE,), jnp.int32),
            pltpu.VMEM((NKEY * _NLANE,), jnp.int32),
            pltpu.VMEM((NKEY,), jnp.int32),
            pltpu.VMEM((NKEY,), jnp.int32),
            pltpu.VMEM((N,), jnp.int32),
            pltpu.VMEM((N,), jnp.int32),
            pltpu.VMEM((N,), jnp.int32),
            pltpu.VMEM((N,), jnp.int32),
            pltpu.VMEM((128, DP), jnp.float32),
            pltpu.VMEM((128, DP), jnp.float32),
            pltpu.SemaphoreType.DMA,
            pltpu.SemaphoreType.DMA,
        ],
    )
    return f(bk2, qkv_t)


NG = 32
GROUPS = NC // NG


def _attn_body(qkv_ref, qkvp_ref, t_ref, tp_ref, out_ref):
    cur = qkv_ref[0][:, :, 0:D]
    vcur = qkv_ref[0][:, :, D:DP]
    prevg = qkvp_ref[0][:, :, 0:D]
    vprevg = qkvp_ref[0][:, :, D:DP]
    tcur = t_ref[0]
    tprevg = tp_ref[0][NG - 1:NG]

    def norm(x):
        var = jnp.mean(x * x, axis=-1, keepdims=True)
        return x * jax.lax.rsqrt(var + 1e-6) * np.float32(1.0 / np.sqrt(D))

    kcur = norm(cur)
    kprev = jnp.concatenate([norm(prevg), kcur[: NG - 1]], axis=0)
    vprev = jnp.concatenate([vprevg, vcur[: NG - 1]], axis=0)
    tprev = jnp.concatenate([tprevg, tcur[: NG - 1]], axis=0)

    k2 = jnp.concatenate([kprev, kcur], axis=1)
    v2 = jnp.concatenate([vprev, vcur], axis=1)
    t2 = jnp.concatenate([tprev, tcur], axis=1)

    dots = jax.lax.dot_general(cur, k2, (((2,), (2,)), ((0,), (0,))),
                               preferred_element_type=jnp.float32)
    tq = tcur[:, :, None]
    tk = t2[:, None, :]
    dots = dots - (tq < tk).astype(jnp.float32) * 1e9 \
                - (tq == tk).astype(jnp.float32) * 1e5
    mx = jnp.max(dots, axis=-1)
    p = jnp.exp(dots - mx[:, :, None])
    ssum = jnp.sum(p, axis=-1)
    lg = mx + jnp.log(ssum)
    o = jax.lax.dot_general(p, v2, (((2,), (1,)), ((0,), (0,))),
                            preferred_element_type=jnp.float32)
    out_ref[0, :, :, 0:D] = o / ssum[:, :, None]
    out_ref[0, :, :, D:DP] = jnp.broadcast_to(lg[:, :, None], (NG, CL, D))


def _attention(qkvs, t_s):
    grid = (B * H, GROUPS)
    blk = lambda p, g: (p, g, 0, 0)
    blkp = lambda p, g: (p, (g * NG - 1) % NC, 0, 0)
    blk3 = lambda p, g: (p, g, 0)
    blk3p = lambda p, g: (p, (g + GROUPS - 1) % GROUPS, 0)
    out = pl.pallas_call(
        _attn_body,
        grid=grid,
        in_specs=[
            pl.BlockSpec((1, NG, CL, DP), blk),
            pl.BlockSpec((1, 1, CL, DP), blkp),
            pl.BlockSpec((1, NG, CL), blk3),
            pl.BlockSpec((1, NG, CL), blk3p),
        ],
        out_specs=pl.BlockSpec((1, NG, CL, DP), blk),
        out_shape=jax.ShapeDtypeStruct((B * H, NC, CL, DP), jnp.float32),
        compiler_params=pltpu.CompilerParams(
            dimension_semantics=("parallel", "arbitrary"),
        ),
    )(qkvs, qkvs, t_s, t_s)
    return out


def _sc_unsort_body(outs_hbm, invp_hbm, outu_hbm,
                    invp_v, sidx_v, rows_v, sem):
    wid = lax.axis_index("s") * 2 + lax.axis_index("c")
    lane = lax.iota(jnp.int32, _NLANE)

    def do_pair(pp, _c):
        p = wid * 2 + pp
        pltpu.sync_copy(invp_hbm.at[p], invp_v)

        def pa(s_, _):
            iv = invp_v[pl.ds(s_ * 16, 16)]
            plsc.store_scatter(
                sidx_v,
                [jnp.full((16,), s_ // 8, jnp.int32), (s_ % 8) * 16 + lane],
                iv + p * N)
            return 0
        lax.fori_loop(0, N // 16, pa, 0)

        def pb(j, _):
            pltpu.sync_copy(outs_hbm.at[pl.ds(p * N + j * 128, 128)], rows_v)
            pltpu.async_copy(rows_v, outu_hbm.at[sidx_v.at[j]], sem).wait()
            return 0
        lax.fori_loop(0, N // 128, pb, 0)
        return 0

    lax.fori_loop(0, 2, do_pair, 0)


def _sc_unsort(outs, invp):
    f = pl.kernel(
        _sc_unsort_body,
        out_type=jax.ShapeDtypeStruct((B * H * N, DP), jnp.float32),
        mesh=plsc.VectorSubcoreMesh(core_axis_name="c", subcore_axis_name="s"),
        compiler_params=pltpu.CompilerParams(needs_layout_passes=False),
        scratch_types=[
            pltpu.VMEM((N,), jnp.int32),
            pltpu.VMEM((N // 128, 128), jnp.int32),
            pltpu.VMEM((128, DP), jnp.float32),
            pltpu.SemaphoreType.DMA,
        ],
    )
    return f(outs, invp)


_TBC = 128


def _comb_body(ou_ref, out_ref):
    ou = ou_ref[0]
    o0 = ou[:, 0, :, 0:D]
    o1 = ou[:, 1, :, 0:D]
    lg0 = ou[:, 0, :, D]
    lg1 = ou[:, 1, :, D]
    m = jnp.maximum(lg0, lg1)
    e0 = jnp.exp(lg0 - m)
    e1 = jnp.exp(lg1 - m)
    w0 = (e0 / (e0 + e1))[:, :, None]
    w1 = (e1 / (e0 + e1))[:, :, None]
    o = o0 * w0 + o1 * w1
    out_ref[0] = o.transpose(1, 0, 2).reshape(_TBC, H * D)


def _combine(outu):
    grid = (B, S // _TBC)
    return pl.pallas_call(
        _comb_body,
        grid=grid,
        in_specs=[
            pl.BlockSpec((1, H, NH, _TBC, DP), lambda b, t: (b, 0, 0, t, 0)),
        ],
        out_specs=pl.BlockSpec((1, _TBC, H * D), lambda b, t: (b, t, 0)),
        out_shape=jax.ShapeDtypeStruct((B, S, H * D), jnp.float32),
        compiler_params=pltpu.CompilerParams(
            dimension_semantics=("parallel", "parallel"),
        ),
    )(outu)


def kernel(hidden_states, Wqk, Wv):
    qkv = _projections(hidden_states, Wqk, Wv)
    buckets = _hash_buckets(qkv)

    bk2 = buckets.reshape(B * H, N)
    qkv_t = qkv.reshape(B * S * H, DP)
    qkvs, ts, invp = _sc_sort_gather(bk2, qkv_t)

    outs = _attention(
        qkvs.reshape(B * H, NC, CL, DP),
        ts.reshape(B * H, NC, CL),
    )

    outu = _sc_unsort(outs.reshape(B * H * N, DP), invp)
    return _combine(outu.reshape(B, H, NH, S, DP))
